# Initial kernel scaffold; baseline (speedup 1.0000x reference)
import jax
import jax.numpy as jnp
from jax.experimental import pallas as pl


def kernel(z_batch, codebook, iter):
    raise NotImplementedError("write your pallas kernel here")


# speedup vs baseline: 2.6027x; 2.6027x over previous
import functools

import jax
import jax.numpy as jnp
from jax.experimental import pallas as pl
from jax.experimental.pallas import tpu as pltpu

B = 64
L = 128
K = 8192
UNROLL = 8


def _vq_body(z_ref, cbt_ref, zq_ref, idx_ref, loss_ref):
    z = z_ref[:]

    def step(w, carry):
        acc_d, acc_i, acc_c = carry
        chunk = cbt_ref[pl.ds(w * UNROLL, UNROLL), :]
        for s in range(UNROLL):
            c = chunk[s : s + 1, :]
            d = jnp.abs(z - c)
            k = w * UNROLL + s
            pred = d < acc_d
            acc_d = jnp.where(pred, d, acc_d)
            acc_i = jnp.where(pred, k, acc_i)
            acc_c = jnp.where(pred, c, acc_c)
        return acc_d, acc_i, acc_c

    init = (
        jnp.full((B, L), jnp.inf, dtype=jnp.float32),
        jnp.zeros((B, L), dtype=jnp.int32),
        jnp.zeros((B, L), dtype=jnp.float32),
    )
    acc_d, acc_i, acc_c = jax.lax.fori_loop(0, K // UNROLL, step, init)

    zq_ref[:] = acc_c
    idx_ref[:] = acc_i
    loss_ref[:] = (1.25 * jnp.mean(acc_d * acc_d)).reshape(1, 1)


@functools.partial(jax.jit, static_argnames=())
def _vq(z_batch, cbt):
    zq, idx, loss = pl.pallas_call(
        _vq_body,
        out_shape=(
            jax.ShapeDtypeStruct((B, L), jnp.float32),
            jax.ShapeDtypeStruct((B, L), jnp.int32),
            jax.ShapeDtypeStruct((1, 1), jnp.float32),
        ),
    )(z_batch, cbt)
    return zq, idx, loss


def kernel(z_batch, codebook, iter):
    cbt = codebook.T
    zq, idx, loss = _vq(z_batch, cbt)
    z_q_st = z_batch + jax.lax.stop_gradient(zq - z_batch)
    return (z_q_st, loss[0, 0], idx)
